# baseline (speedup 1.0000x reference)
import functools

import jax
import jax.numpy as jnp
from jax import lax
from jax.experimental import pallas as pl
from jax.experimental.pallas import tpu as pltpu
from jax.experimental.pallas import tpu_sc as plsc

NC = 2
NS = 16
NW = NC * NS
LANES = 16
C = 128


def _transform_body(h_ref, w_ref, o_ref):
    o_ref[0] = jnp.dot(h_ref[...], w_ref[0], preferred_element_type=jnp.float32)


def _tc_transform(h, w_all, nb):
    n, d = h.shape
    rp1 = w_all.shape[0]
    return pl.pallas_call(
        _transform_body,
        grid=(n // nb, rp1),
        in_specs=[
            pl.BlockSpec((nb, d), lambda i, r: (i, 0)),
            pl.BlockSpec((1, d, d), lambda i, r: (r, 0, 0)),
        ],
        out_specs=pl.BlockSpec((1, nb, d), lambda i, r: (r, i, 0)),
        out_shape=jax.ShapeDtypeStruct((rp1, n, d), jnp.float32),
    )(h, w_all)


def _transform2_body(r_ref, p0_ref, p1_ref, b_ref, w_ref, o_ref):
    h = jax.nn.relu(r_ref[...] + p0_ref[...] + p1_ref[...] + b_ref[...])
    o_ref[0] = jnp.dot(h, w_ref[0], preferred_element_type=jnp.float32)


def _tc_transform2(root, p0, p1, b, w_all, nb):
    n, d = root.shape
    rp1 = w_all.shape[0]
    return pl.pallas_call(
        _transform2_body,
        grid=(n // nb, rp1),
        in_specs=[
            pl.BlockSpec((nb, d), lambda i, r: (i, 0)),
            pl.BlockSpec((nb, d), lambda i, r: (i, 0)),
            pl.BlockSpec((nb, d), lambda i, r: (i, 0)),
            pl.BlockSpec((1, d), lambda i, r: (0, 0)),
            pl.BlockSpec((1, d, d), lambda i, r: (r, 0, 0)),
        ],
        out_specs=pl.BlockSpec((1, nb, d), lambda i, r: (r, i, 0)),
        out_shape=jax.ShapeDtypeStruct((rp1, n, d), jnp.float32),
    )(root, p0, p1, b, w_all)


def _combine2_body(r_ref, p0_ref, p1_ref, b_ref, wo_ref, bo_ref, o_ref):
    h = jax.nn.relu(r_ref[...] + p0_ref[...] + p1_ref[...] + b_ref[...])
    logit = jnp.dot(h, wo_ref[...], preferred_element_type=jnp.float32)
    o_ref[...] = jax.nn.sigmoid(logit[:, 0] + bo_ref[0, 0])


def _tc_combine2(root, p0, p1, b, w_out, b_out):
    n, d = root.shape
    return pl.pallas_call(
        _combine2_body,
        out_shape=jax.ShapeDtypeStruct((n,), jnp.float32),
    )(root, p0, p1, b, w_out, b_out)


def _sc_mesh():
    return plsc.VectorSubcoreMesh(core_axis_name="c", subcore_axis_name="s")


def _make_counts(nrp, kc, r):
    piece = 1024
    per_tile = nrp // NS
    npiece = per_tile // piece

    @functools.partial(
        pl.kernel,
        out_type=jax.ShapeDtypeStruct((NC * nrp,), jnp.float32),
        mesh=_sc_mesh(),
        scratch_types=[
            pltpu.VMEM((kc, C), jnp.int32),
            pltpu.VMEM((kc, C), jnp.int32),
            pltpu.VMEM((kc, C), jnp.float32),
            pltpu.VMEM((kc, C), jnp.int32),
            pltpu.VMEM((piece,), jnp.float32),
            pltpu.VMEM_SHARED((nrp,), jnp.float32),
        ],
    )
    def counts(dst_hbm, type_hbm, valid_hbm, cnt_hbm, dbuf, tbuf, vbuf, ibuf,
               zbuf, acc):
        cid = lax.axis_index("c")
        sid = lax.axis_index("s")
        wid = cid * NS + sid
        pltpu.sync_copy(dst_hbm.at[wid], dbuf)
        pltpu.sync_copy(type_hbm.at[wid], tbuf)
        pltpu.sync_copy(valid_hbm.at[wid], vbuf)

        def idx_body(k, _):
            for j in range(C // LANES):
                sl = pl.ds(j * LANES, LANES)
                ibuf[k, sl] = dbuf[k, sl] * r + tbuf[k, sl]
            return 0
        lax.fori_loop(0, kc, idx_body, 0)

        def z_body(i, _):
            zbuf[pl.ds(i * LANES, LANES)] = jnp.zeros((LANES,), jnp.float32)
            return 0
        lax.fori_loop(0, piece // LANES, z_body, 0)
        for p in range(npiece):
            pltpu.sync_copy(zbuf,
                            acc.at[pl.ds(sid * per_tile + p * piece, piece)])
        plsc.subcore_barrier()

        def sc_body(k, _):
            pltpu.sync_copy(vbuf.at[k], acc.at[ibuf.at[k]], add=True)
            return 0
        lax.fori_loop(0, kc, sc_body, 0)
        plsc.subcore_barrier()

        for p in range(npiece):
            off = sid * per_tile + p * piece
            pltpu.sync_copy(acc.at[pl.ds(off, piece)], zbuf)
            pltpu.sync_copy(zbuf, cnt_hbm.at[pl.ds(cid * nrp + off, piece)])

    return counts


def _make_weights(n, kc, r, nrp):
    WV = 5
    assert kc % WV == 0

    @functools.partial(
        pl.kernel,
        out_type=(jax.ShapeDtypeStruct((NW, kc * C), jnp.float32),
                  jax.ShapeDtypeStruct((NW, kc, C), jnp.int32)),
        mesh=_sc_mesh(),
        scratch_types=[
            pltpu.VMEM((kc, C), jnp.int32),
            pltpu.VMEM((kc, C), jnp.int32),
            pltpu.VMEM((kc, C), jnp.int32),
            pltpu.VMEM((kc, C), jnp.float32),
            pltpu.VMEM((kc, C), jnp.int32),
            pltpu.VMEM((kc, C), jnp.int32),
            pltpu.VMEM((kc * C,), jnp.float32),
            pltpu.VMEM((WV, C), jnp.float32),
            pltpu.VMEM((WV, C), jnp.float32),
            pltpu.SemaphoreType.DMA,
        ],
    )
    def weights(src_hbm, dst_hbm, type_hbm, valid_hbm, cnt0_hbm, cnt1_hbm,
                w_hbm, g_hbm, sbuf, dbuf, tbuf, vbuf, ibuf, gbuf,
                wbuf, c0buf, c1buf, sem):
        cid = lax.axis_index("c")
        sid = lax.axis_index("s")
        wid = cid * NS + sid
        pltpu.sync_copy(src_hbm.at[wid], sbuf)
        pltpu.sync_copy(dst_hbm.at[wid], dbuf)
        pltpu.sync_copy(type_hbm.at[wid], tbuf)
        pltpu.sync_copy(valid_hbm.at[wid], vbuf)

        def idx_body(k, _):
            for j in range(C // LANES):
                sl = pl.ds(j * LANES, LANES)
                t = tbuf[k, sl]
                ibuf[k, sl] = dbuf[k, sl] * r + t
                gbuf[k, sl] = t * n + sbuf[k, sl]
            return 0
        lax.fori_loop(0, kc, idx_body, 0)

        def wave_body(w, _):
            k0 = w * WV
            for j in range(WV):
                pltpu.async_copy(cnt0_hbm.at[ibuf.at[k0 + j]],
                                 c0buf.at[j], sem)
                pltpu.async_copy(cnt1_hbm.at[ibuf.at[k0 + j]],
                                 c1buf.at[j], sem)
            for j in range(WV):
                pltpu.make_async_copy(cnt0_hbm.at[ibuf.at[k0 + j]],
                                      c0buf.at[j], sem).wait()
                pltpu.make_async_copy(cnt1_hbm.at[ibuf.at[k0 + j]],
                                      c1buf.at[j], sem).wait()
            for j in range(WV):
                for g in range(C // LANES):
                    sl = pl.ds(g * LANES, LANES)
                    cs = c0buf[j, sl] + c1buf[j, sl]
                    wbuf[pl.ds((k0 + j) * C + g * LANES, LANES)] = (
                        vbuf[k0 + j, sl] / jnp.maximum(cs, 1.0))
            return 0
        lax.fori_loop(0, kc // WV, wave_body, 0)

        pltpu.sync_copy(wbuf, w_hbm.at[wid])
        pltpu.sync_copy(gbuf, g_hbm.at[wid])

    return weights


def _make_edge_pass(npad, d, kc):
    rows_per_tile = npad // NS
    piece = 32
    npiece = rows_per_tile // piece
    peel = 2 - (kc % 2)

    @functools.partial(
        pl.kernel,
        out_type=jax.ShapeDtypeStruct((NC, npad, d), jnp.float32),
        mesh=_sc_mesh(),
        scratch_types=[
            pltpu.VMEM((3, C), jnp.int32),
            pltpu.VMEM((3, C), jnp.int32),
            pltpu.VMEM((C, d), jnp.float32),
            pltpu.VMEM((C, d), jnp.float32),
            pltpu.VMEM((C,), jnp.int32),
            pltpu.VMEM((C,), jnp.int32),
            pltpu.VMEM((piece, d), jnp.float32),
            pltpu.VMEM_SHARED((npad, d), jnp.float32),
            pltpu.SemaphoreType.DMA,
            pltpu.SemaphoreType.DMA,
            pltpu.SemaphoreType.DMA,
            pltpu.SemaphoreType.DMA,
            pltpu.SemaphoreType.DMA,
            pltpu.SemaphoreType.DMA,
        ],
    )
    def edge_pass(h_hbm, edata_hbm, part_hbm, ebuf0, ebuf1, rows0, rows1,
                  di0, di1, zrow, acc, semg0, semg1, sems0, sems1, seme0,
                  seme1):
        cid = lax.axis_index("c")
        sid = lax.axis_index("s")
        wid = cid * NS + sid
        ebufs = (ebuf0, ebuf1)
        rows = (rows0, rows1)
        dis = (di0, di1)
        semg = (semg0, semg1)
        sems = (sems0, sems1)
        seme = (seme0, seme1)

        def z_body(i, _):
            for j in range(d // LANES):
                zrow[i, pl.ds(j * LANES, LANES)] = jnp.zeros((LANES,), jnp.float32)
            return 0
        lax.fori_loop(0, piece, z_body, 0)
        for p in range(npiece):
            pltpu.sync_copy(
                zrow, acc.at[pl.ds(sid * rows_per_tile + p * piece, piece)])
        plsc.subcore_barrier()

        def sub_iter(k, p, first):
            q = 1 - p
            eb, rw, di = ebufs[p], rows[p], dis[p]
            ebq, rwq, diq = ebufs[q], rows[q], dis[q]
            pltpu.make_async_copy(h_hbm.at[eb.at[0]], rw, semg[p]).wait()
            if not first:
                pltpu.make_async_copy(rwq, acc.at[diq], sems[q]).wait()
            pltpu.make_async_copy(edata_hbm.at[wid, k + 1], ebq,
                                  seme[q]).wait()
            pltpu.async_copy(h_hbm.at[ebq.at[0]], rwq, semg[q])
            for g in range(C // LANES):
                sl = pl.ds(g * LANES, LANES)
                di[sl] = eb[1, sl]
                wv = lax.bitcast_convert_type(eb[2, sl], jnp.float32)
                for t in range(LANES):
                    wt = lax.gather(
                        wv, jnp.full((LANES, 1), t, jnp.int32),
                        lax.GatherDimensionNumbers(
                            offset_dims=(), collapsed_slice_dims=(0,),
                            start_index_map=(0,)),
                        slice_sizes=(1,),
                        mode=lax.GatherScatterMode.PROMISE_IN_BOUNDS)
                    e = g * LANES + t
                    for j in range(d // LANES):
                        slj = pl.ds(j * LANES, LANES)
                        rw[e, slj] = rw[e, slj] * wt
            pltpu.async_copy(rw, acc.at[di], sems[p], add=True)
            pltpu.async_copy(edata_hbm.at[wid, k + 2], eb, seme[p])

        pltpu.sync_copy(edata_hbm.at[wid, 0], ebuf0)
        pltpu.async_copy(h_hbm.at[ebuf0.at[0]], rows0, semg0)
        pltpu.async_copy(edata_hbm.at[wid, 1], ebuf1, seme1)
        for k in range(peel):
            sub_iter(k, k % 2, k == 0)

        def body(i, _):
            k = peel + 2 * i
            sub_iter(k, peel % 2, False)
            sub_iter(k + 1, (peel + 1) % 2, False)
            return 0
        lax.fori_loop(0, (kc - peel) // 2, body, 0)

        pg = kc % 2
        pl_ = (kc - 1) % 2
        pltpu.make_async_copy(h_hbm.at[ebufs[pg].at[0]], rows[pg],
                              semg[pg]).wait()
        pltpu.make_async_copy(rows[pl_], acc.at[dis[pl_]], sems[pl_]).wait()
        pltpu.make_async_copy(edata_hbm.at[wid, kc + 1], ebufs[pl_],
                              seme[pl_]).wait()
        plsc.subcore_barrier()

        for p in range(npiece):
            sl = pl.ds(sid * rows_per_tile + p * piece, piece)
            pltpu.sync_copy(acc.at[sl], zrow)
            pltpu.sync_copy(zrow, part_hbm.at[cid, sl])

    return edge_pass


def kernel(x, edge_index, edge_type, W_rel1, W_root1, b1, W_rel2, W_root2,
           b2, W_out, b_out):
    n, d = x.shape
    r = W_rel1.shape[0]
    e = edge_type.shape[0]
    nr = n * r

    step = NW * C * 5
    e_pad = ((e + step - 1) // step) * step
    kc = e_pad // (NW * C)

    src = edge_index[0]
    dst = edge_index[1]
    valid = jnp.ones((e,), jnp.float32)
    if e_pad != e:
        pad = e_pad - e
        src = jnp.concatenate([src, jnp.zeros((pad,), src.dtype)])
        dst = jnp.concatenate([dst, jnp.zeros((pad,), dst.dtype)])
        edge_type = jnp.concatenate([edge_type, jnp.zeros((pad,), edge_type.dtype)])
        valid = jnp.concatenate([valid, jnp.zeros((pad,), jnp.float32)])
    src_r = src.reshape(NW, kc, C)
    dst_r = dst.reshape(NW, kc, C)
    type_r = edge_type.reshape(NW, kc, C)
    valid_r = valid.reshape(NW, kc, C)

    binstep = NS * 1024
    nrp = ((nr + binstep - 1) // binstep) * binstep
    nodestep = NS * 128
    npad = ((n + nodestep - 1) // nodestep) * nodestep

    cnt = _make_counts(nrp, kc, r)(dst_r, type_r, valid_r)
    w_r, g_r = _make_weights(n, kc, r, nrp)(src_r, dst_r, type_r, valid_r,
                                            cnt[:nrp], cnt[nrp:])

    edata = jnp.stack(
        [g_r, dst_r, lax.bitcast_convert_type(w_r.reshape(NW, kc, C), jnp.int32)],
        axis=2)
    edata = jnp.pad(edata, ((0, 0), (0, 2), (0, 0), (0, 0)))

    edge_pass = _make_edge_pass(npad, d, kc)
    nb = 1000

    w_all1 = jnp.concatenate([W_rel1, W_root1[None]], axis=0)
    h_all1 = _tc_transform(x, w_all1, nb)
    part1 = edge_pass(h_all1[:r].reshape(r * n, d), edata)

    w_all2 = jnp.concatenate([W_rel2, W_root2[None]], axis=0)
    h_all2 = _tc_transform2(h_all1[r], part1[0, :n], part1[1, :n],
                            b1.reshape(1, d), w_all2, nb)
    part2 = edge_pass(h_all2[:r].reshape(r * n, d), edata)
    return _tc_combine2(h_all2[r], part2[0, :n], part2[1, :n],
                        b2.reshape(1, d), W_out, b_out.reshape(1, 1))


# speedup vs baseline: 1.9444x; 1.9444x over previous
import functools

import jax
import jax.numpy as jnp
from jax import lax
from jax.experimental import pallas as pl
from jax.experimental.pallas import tpu as pltpu
from jax.experimental.pallas import tpu_sc as plsc

NC = 2
NS = 16
NW = NC * NS
LANES = 16
C = 80


def _transform_body(h_ref, w_ref, o_ref):
    o_ref[0] = jnp.dot(h_ref[...], w_ref[0], preferred_element_type=jnp.float32)


def _tc_transform(h, w_all, nb):
    n, d = h.shape
    rp1 = w_all.shape[0]
    return pl.pallas_call(
        _transform_body,
        grid=(n // nb, rp1),
        in_specs=[
            pl.BlockSpec((nb, d), lambda i, r: (i, 0)),
            pl.BlockSpec((1, d, d), lambda i, r: (r, 0, 0)),
        ],
        out_specs=pl.BlockSpec((1, nb, d), lambda i, r: (r, i, 0)),
        out_shape=jax.ShapeDtypeStruct((rp1, n, d), jnp.float32),
    )(h, w_all)


def _transform2_body(r_ref, p0_ref, p1_ref, b_ref, w_ref, o_ref):
    h = jax.nn.relu(r_ref[...] + p0_ref[...] + p1_ref[...] + b_ref[...])
    o_ref[0] = jnp.dot(h, w_ref[0], preferred_element_type=jnp.float32)


def _tc_transform2(root, p0, p1, b, w_all, nb):
    n, d = root.shape
    rp1 = w_all.shape[0]
    return pl.pallas_call(
        _transform2_body,
        grid=(n // nb, rp1),
        in_specs=[
            pl.BlockSpec((nb, d), lambda i, r: (i, 0)),
            pl.BlockSpec((nb, d), lambda i, r: (i, 0)),
            pl.BlockSpec((nb, d), lambda i, r: (i, 0)),
            pl.BlockSpec((1, d), lambda i, r: (0, 0)),
            pl.BlockSpec((1, d, d), lambda i, r: (r, 0, 0)),
        ],
        out_specs=pl.BlockSpec((1, nb, d), lambda i, r: (r, i, 0)),
        out_shape=jax.ShapeDtypeStruct((rp1, n, d), jnp.float32),
    )(root, p0, p1, b, w_all)


def _combine2_body(r_ref, p0_ref, p1_ref, b_ref, wo_ref, bo_ref, o_ref):
    h = jax.nn.relu(r_ref[...] + p0_ref[...] + p1_ref[...] + b_ref[...])
    logit = jnp.dot(h, wo_ref[...], preferred_element_type=jnp.float32)
    o_ref[...] = jax.nn.sigmoid(logit[:, 0] + bo_ref[0, 0])


def _tc_combine2(root, p0, p1, b, w_out, b_out):
    n, d = root.shape
    return pl.pallas_call(
        _combine2_body,
        out_shape=jax.ShapeDtypeStruct((n,), jnp.float32),
    )(root, p0, p1, b, w_out, b_out)


def _sc_mesh():
    return plsc.VectorSubcoreMesh(core_axis_name="c", subcore_axis_name="s")


def _make_counts(nrp, kc, r):
    piece = 1024
    per_tile = nrp // NS
    npiece = per_tile // piece

    @functools.partial(
        pl.kernel,
        out_type=jax.ShapeDtypeStruct((NC * nrp,), jnp.float32),
        mesh=_sc_mesh(),
        scratch_types=[
            pltpu.VMEM((kc, C), jnp.int32),
            pltpu.VMEM((kc, C), jnp.int32),
            pltpu.VMEM((kc, C), jnp.float32),
            pltpu.VMEM((kc, C), jnp.int32),
            pltpu.VMEM((piece,), jnp.float32),
            pltpu.VMEM_SHARED((nrp,), jnp.float32),
        ],
    )
    def counts(dst_hbm, type_hbm, valid_hbm, cnt_hbm, dbuf, tbuf, vbuf, ibuf,
               zbuf, acc):
        cid = lax.axis_index("c")
        sid = lax.axis_index("s")
        wid = cid * NS + sid
        pltpu.sync_copy(dst_hbm.at[wid], dbuf)
        pltpu.sync_copy(type_hbm.at[wid], tbuf)
        pltpu.sync_copy(valid_hbm.at[wid], vbuf)

        def idx_body(k, _):
            for j in range(C // LANES):
                sl = pl.ds(j * LANES, LANES)
                ibuf[k, sl] = dbuf[k, sl] * r + tbuf[k, sl]
            return 0
        lax.fori_loop(0, kc, idx_body, 0)

        def z_body(i, _):
            zbuf[pl.ds(i * LANES, LANES)] = jnp.zeros((LANES,), jnp.float32)
            return 0
        lax.fori_loop(0, piece // LANES, z_body, 0)
        for p in range(npiece):
            pltpu.sync_copy(zbuf,
                            acc.at[pl.ds(sid * per_tile + p * piece, piece)])
        plsc.subcore_barrier()

        def sc_body(k, _):
            pltpu.sync_copy(vbuf.at[k], acc.at[ibuf.at[k]], add=True)
            return 0
        lax.fori_loop(0, kc, sc_body, 0)
        plsc.subcore_barrier()

        for p in range(npiece):
            off = sid * per_tile + p * piece
            pltpu.sync_copy(acc.at[pl.ds(off, piece)], zbuf)
            pltpu.sync_copy(zbuf, cnt_hbm.at[pl.ds(cid * nrp + off, piece)])

    return counts


def _make_weights(n, kc, r, nrp):
    WV = 5
    assert kc % WV == 0

    @functools.partial(
        pl.kernel,
        out_type=(jax.ShapeDtypeStruct((NW, kc * C), jnp.float32),
                  jax.ShapeDtypeStruct((NW, kc, C), jnp.int32)),
        mesh=_sc_mesh(),
        scratch_types=[
            pltpu.VMEM((kc, C), jnp.int32),
            pltpu.VMEM((kc, C), jnp.int32),
            pltpu.VMEM((kc, C), jnp.int32),
            pltpu.VMEM((kc, C), jnp.float32),
            pltpu.VMEM((kc, C), jnp.int32),
            pltpu.VMEM((kc, C), jnp.int32),
            pltpu.VMEM((kc * C,), jnp.float32),
            pltpu.VMEM((WV, C), jnp.float32),
            pltpu.VMEM((WV, C), jnp.float32),
            pltpu.SemaphoreType.DMA,
        ],
    )
    def weights(src_hbm, dst_hbm, type_hbm, valid_hbm, cnt0_hbm, cnt1_hbm,
                w_hbm, g_hbm, sbuf, dbuf, tbuf, vbuf, ibuf, gbuf,
                wbuf, c0buf, c1buf, sem):
        cid = lax.axis_index("c")
        sid = lax.axis_index("s")
        wid = cid * NS + sid
        pltpu.sync_copy(src_hbm.at[wid], sbuf)
        pltpu.sync_copy(dst_hbm.at[wid], dbuf)
        pltpu.sync_copy(type_hbm.at[wid], tbuf)
        pltpu.sync_copy(valid_hbm.at[wid], vbuf)

        def idx_body(k, _):
            for j in range(C // LANES):
                sl = pl.ds(j * LANES, LANES)
                t = tbuf[k, sl]
                ibuf[k, sl] = dbuf[k, sl] * r + t
                gbuf[k, sl] = t * n + sbuf[k, sl]
            return 0
        lax.fori_loop(0, kc, idx_body, 0)

        def wave_body(w, _):
            k0 = w * WV
            for j in range(WV):
                pltpu.async_copy(cnt0_hbm.at[ibuf.at[k0 + j]],
                                 c0buf.at[j], sem)
                pltpu.async_copy(cnt1_hbm.at[ibuf.at[k0 + j]],
                                 c1buf.at[j], sem)
            for j in range(WV):
                pltpu.make_async_copy(cnt0_hbm.at[ibuf.at[k0 + j]],
                                      c0buf.at[j], sem).wait()
                pltpu.make_async_copy(cnt1_hbm.at[ibuf.at[k0 + j]],
                                      c1buf.at[j], sem).wait()
            for j in range(WV):
                for g in range(C // LANES):
                    sl = pl.ds(g * LANES, LANES)
                    cs = c0buf[j, sl] + c1buf[j, sl]
                    wbuf[pl.ds((k0 + j) * C + g * LANES, LANES)] = (
                        vbuf[k0 + j, sl] / jnp.maximum(cs, 1.0))
            return 0
        lax.fori_loop(0, kc // WV, wave_body, 0)

        pltpu.sync_copy(wbuf, w_hbm.at[wid])
        pltpu.sync_copy(gbuf, g_hbm.at[wid])

    return weights


def _make_edge_pass(npad, d, kc):
    rows_per_tile = npad // NS
    piece = 32
    npiece = rows_per_tile // piece
    peel = 2 - (kc % 2)

    @functools.partial(
        pl.kernel,
        out_type=jax.ShapeDtypeStruct((NC, npad, d), jnp.float32),
        mesh=_sc_mesh(),
        scratch_types=[
            pltpu.VMEM((3, C), jnp.int32),
            pltpu.VMEM((3, C), jnp.int32),
            pltpu.VMEM((C, d), jnp.float32),
            pltpu.VMEM((C, d), jnp.float32),
            pltpu.VMEM((C,), jnp.int32),
            pltpu.VMEM((C,), jnp.int32),
            pltpu.VMEM((piece, d), jnp.float32),
            pltpu.VMEM_SHARED((npad, d), jnp.float32),
            pltpu.SemaphoreType.DMA,
            pltpu.SemaphoreType.DMA,
            pltpu.SemaphoreType.DMA,
            pltpu.SemaphoreType.DMA,
            pltpu.SemaphoreType.DMA,
            pltpu.SemaphoreType.DMA,
        ],
    )
    def edge_pass(h_hbm, edata_hbm, part_hbm, ebuf0, ebuf1, rows0, rows1,
                  di0, di1, zrow, acc, semg0, semg1, sems0, sems1, seme0,
                  seme1):
        cid = lax.axis_index("c")
        sid = lax.axis_index("s")
        wid = cid * NS + sid
        ebufs = (ebuf0, ebuf1)
        rows = (rows0, rows1)
        dis = (di0, di1)
        semg = (semg0, semg1)
        sems = (sems0, sems1)
        seme = (seme0, seme1)

        def z_body(i, _):
            for j in range(d // LANES):
                zrow[i, pl.ds(j * LANES, LANES)] = jnp.zeros((LANES,), jnp.float32)
            return 0
        lax.fori_loop(0, piece, z_body, 0)
        for p in range(npiece):
            pltpu.sync_copy(
                zrow, acc.at[pl.ds(sid * rows_per_tile + p * piece, piece)])
        plsc.subcore_barrier()

        def sub_iter(k, p, first):
            q = 1 - p
            eb, rw, di = ebufs[p], rows[p], dis[p]
            ebq, rwq, diq = ebufs[q], rows[q], dis[q]
            pltpu.make_async_copy(h_hbm.at[eb.at[0]], rw, semg[p]).wait()
            if not first:
                pltpu.make_async_copy(rwq, acc.at[diq], sems[q]).wait()
            pltpu.make_async_copy(edata_hbm.at[wid, k + 1], ebq,
                                  seme[q]).wait()
            pltpu.async_copy(h_hbm.at[ebq.at[0]], rwq, semg[q])
            for g in range(C // LANES):
                sl = pl.ds(g * LANES, LANES)
                di[sl] = eb[1, sl]
                wv = lax.bitcast_convert_type(eb[2, sl], jnp.float32)
                for t in range(LANES):
                    wt = lax.gather(
                        wv, jnp.full((LANES, 1), t, jnp.int32),
                        lax.GatherDimensionNumbers(
                            offset_dims=(), collapsed_slice_dims=(0,),
                            start_index_map=(0,)),
                        slice_sizes=(1,),
                        mode=lax.GatherScatterMode.PROMISE_IN_BOUNDS)
                    e = g * LANES + t
                    for j in range(d // LANES):
                        slj = pl.ds(j * LANES, LANES)
                        rw[e, slj] = rw[e, slj] * wt
            pltpu.async_copy(rw, acc.at[di], sems[p], add=True)
            pltpu.async_copy(edata_hbm.at[wid, k + 2], eb, seme[p])

        pltpu.sync_copy(edata_hbm.at[wid, 0], ebuf0)
        pltpu.async_copy(h_hbm.at[ebuf0.at[0]], rows0, semg0)
        pltpu.async_copy(edata_hbm.at[wid, 1], ebuf1, seme1)
        for k in range(peel):
            sub_iter(k, k % 2, k == 0)

        def body(i, _):
            k = peel + 2 * i
            sub_iter(k, peel % 2, False)
            sub_iter(k + 1, (peel + 1) % 2, False)
            return 0
        lax.fori_loop(0, (kc - peel) // 2, body, 0)

        pg = kc % 2
        pl_ = (kc - 1) % 2
        pltpu.make_async_copy(h_hbm.at[ebufs[pg].at[0]], rows[pg],
                              semg[pg]).wait()
        pltpu.make_async_copy(rows[pl_], acc.at[dis[pl_]], sems[pl_]).wait()
        pltpu.make_async_copy(edata_hbm.at[wid, kc + 1], ebufs[pl_],
                              seme[pl_]).wait()
        plsc.subcore_barrier()

        for p in range(npiece):
            sl = pl.ds(sid * rows_per_tile + p * piece, piece)
            pltpu.sync_copy(acc.at[sl], zrow)
            pltpu.sync_copy(zrow, part_hbm.at[cid, sl])

    return edge_pass


def kernel(x, edge_index, edge_type, W_rel1, W_root1, b1, W_rel2, W_root2,
           b2, W_out, b_out):
    n, d = x.shape
    r = W_rel1.shape[0]
    e = edge_type.shape[0]
    nr = n * r

    step = NW * C
    e_pad = ((e + step - 1) // step) * step
    kc = e_pad // (NW * C)

    src = edge_index[0]
    dst = edge_index[1]
    valid = jnp.ones((e,), jnp.float32)
    if e_pad != e:
        pad = e_pad - e
        src = jnp.concatenate(
            [src, (jnp.arange(pad, dtype=src.dtype) % n)])
        dst = jnp.concatenate([dst, jnp.zeros((pad,), dst.dtype)])
        edge_type = jnp.concatenate([edge_type, jnp.zeros((pad,), edge_type.dtype)])
        valid = jnp.concatenate([valid, jnp.zeros((pad,), jnp.float32)])
    src_r = src.reshape(NW, kc, C)
    dst_r = dst.reshape(NW, kc, C)
    type_r = edge_type.reshape(NW, kc, C)
    valid_r = valid.reshape(NW, kc, C)

    binstep = NS * 1024
    nrp = ((nr + binstep - 1) // binstep) * binstep
    nodestep = NS * 128
    npad = ((n + nodestep - 1) // nodestep) * nodestep

    cnt = _make_counts(nrp, kc, r)(dst_r, type_r, valid_r)
    w_r, g_r = _make_weights(n, kc, r, nrp)(src_r, dst_r, type_r, valid_r,
                                            cnt[:nrp], cnt[nrp:])

    edata = jnp.stack(
        [g_r, dst_r, lax.bitcast_convert_type(w_r.reshape(NW, kc, C), jnp.int32)],
        axis=2)
    edata = jnp.pad(edata, ((0, 0), (0, 2), (0, 0), (0, 0)))

    edge_pass = _make_edge_pass(npad, d, kc)
    nb = 1000

    w_all1 = jnp.concatenate([W_rel1, W_root1[None]], axis=0)
    h_all1 = _tc_transform(x, w_all1, nb)
    part1 = edge_pass(h_all1[:r].reshape(r * n, d), edata)

    w_all2 = jnp.concatenate([W_rel2, W_root2[None]], axis=0)
    h_all2 = _tc_transform2(h_all1[r], part1[0, :n], part1[1, :n],
                            b1.reshape(1, d), w_all2, nb)
    part2 = edge_pass(h_all2[:r].reshape(r * n, d), edata)
    return _tc_combine2(h_all2[r], part2[0, :n], part2[1, :n],
                        b2.reshape(1, d), W_out, b_out.reshape(1, 1))


# speedup vs baseline: 2.0007x; 1.0289x over previous
import functools

import jax
import jax.numpy as jnp
from jax import lax
from jax.experimental import pallas as pl
from jax.experimental.pallas import tpu as pltpu
from jax.experimental.pallas import tpu_sc as plsc

NC = 2
NS = 16
NW = NC * NS
LANES = 16
C = 80


def _transform_body(h_ref, w_ref, o_ref):
    o_ref[0] = jnp.dot(h_ref[...], w_ref[0], preferred_element_type=jnp.float32)


def _tc_transform(h, w_all, nb):
    n, d = h.shape
    rp1 = w_all.shape[0]
    return pl.pallas_call(
        _transform_body,
        grid=(n // nb, rp1),
        in_specs=[
            pl.BlockSpec((nb, d), lambda i, r: (i, 0)),
            pl.BlockSpec((1, d, d), lambda i, r: (r, 0, 0)),
        ],
        out_specs=pl.BlockSpec((1, nb, d), lambda i, r: (r, i, 0)),
        out_shape=jax.ShapeDtypeStruct((rp1, n, d), jnp.float32),
    )(h, w_all)


def _transform2_body(r_ref, p0_ref, p1_ref, b_ref, w_ref, o_ref):
    h = jax.nn.relu(r_ref[...] + p0_ref[...] + p1_ref[...] + b_ref[...])
    o_ref[0] = jnp.dot(h, w_ref[0], preferred_element_type=jnp.float32)


def _tc_transform2(root, p0, p1, b, w_all, nb):
    n, d = root.shape
    rp1 = w_all.shape[0]
    return pl.pallas_call(
        _transform2_body,
        grid=(n // nb, rp1),
        in_specs=[
            pl.BlockSpec((nb, d), lambda i, r: (i, 0)),
            pl.BlockSpec((nb, d), lambda i, r: (i, 0)),
            pl.BlockSpec((nb, d), lambda i, r: (i, 0)),
            pl.BlockSpec((1, d), lambda i, r: (0, 0)),
            pl.BlockSpec((1, d, d), lambda i, r: (r, 0, 0)),
        ],
        out_specs=pl.BlockSpec((1, nb, d), lambda i, r: (r, i, 0)),
        out_shape=jax.ShapeDtypeStruct((rp1, n, d), jnp.float32),
    )(root, p0, p1, b, w_all)


def _combine2_body(r_ref, p0_ref, p1_ref, b_ref, wo_ref, bo_ref, o_ref):
    h = jax.nn.relu(r_ref[...] + p0_ref[...] + p1_ref[...] + b_ref[...])
    logit = jnp.dot(h, wo_ref[...], preferred_element_type=jnp.float32)
    o_ref[...] = jax.nn.sigmoid(logit[:, 0] + bo_ref[0, 0])


def _tc_combine2(root, p0, p1, b, w_out, b_out):
    n, d = root.shape
    return pl.pallas_call(
        _combine2_body,
        out_shape=jax.ShapeDtypeStruct((n,), jnp.float32),
    )(root, p0, p1, b, w_out, b_out)


def _sc_mesh():
    return plsc.VectorSubcoreMesh(core_axis_name="c", subcore_axis_name="s")


def _make_counts_weights(n, nrp, kc, r):
    piece = 1024
    per_tile = nrp // NS
    npiece = per_tile // piece

    @functools.partial(
        pl.kernel,
        out_type=(jax.ShapeDtypeStruct((NW, kc * C), jnp.float32),
                  jax.ShapeDtypeStruct((NW, kc, C), jnp.int32)),
        mesh=_sc_mesh(),
        scratch_types=[
            pltpu.VMEM((kc, C), jnp.int32),
            pltpu.VMEM((kc, C), jnp.int32),
            pltpu.VMEM((kc, C), jnp.int32),
            pltpu.VMEM((kc, C), jnp.float32),
            pltpu.VMEM((kc, C), jnp.int32),
            pltpu.VMEM((kc, C), jnp.int32),
            pltpu.VMEM((kc * C,), jnp.float32),
            pltpu.VMEM((C,), jnp.float32),
            pltpu.VMEM((piece,), jnp.float32),
            pltpu.VMEM_SHARED((nrp,), jnp.float32),
        ],
    )
    def counts_weights(src_hbm, dst_hbm, type_hbm, valid_hbm,
                       w_hbm, g_hbm, sbuf, dbuf, tbuf, vbuf, ibuf, gbuf,
                       wbuf, cbuf, zbuf, acc):
        cid = lax.axis_index("c")
        sid = lax.axis_index("s")
        wid_own = cid * NS + sid
        wid_other = (1 - cid) * NS + sid

        def z_body(i, _):
            zbuf[pl.ds(i * LANES, LANES)] = jnp.zeros((LANES,), jnp.float32)
            return 0
        lax.fori_loop(0, piece // LANES, z_body, 0)
        for p in range(npiece):
            pltpu.sync_copy(zbuf,
                            acc.at[pl.ds(sid * per_tile + p * piece, piece)])
        plsc.subcore_barrier()

        for wid in (wid_own, wid_other):
            pltpu.sync_copy(dst_hbm.at[wid], dbuf)
            pltpu.sync_copy(type_hbm.at[wid], tbuf)
            pltpu.sync_copy(valid_hbm.at[wid], vbuf)

            def idx_body(k, _):
                for j in range(C // LANES):
                    sl = pl.ds(j * LANES, LANES)
                    ibuf[k, sl] = dbuf[k, sl] * r + tbuf[k, sl]
                return 0
            lax.fori_loop(0, kc, idx_body, 0)

            def sc_body(k, _):
                pltpu.sync_copy(vbuf.at[k], acc.at[ibuf.at[k]], add=True)
                return 0
            lax.fori_loop(0, kc, sc_body, 0)
        plsc.subcore_barrier()

        pltpu.sync_copy(src_hbm.at[wid_own], sbuf)
        pltpu.sync_copy(dst_hbm.at[wid_own], dbuf)
        pltpu.sync_copy(type_hbm.at[wid_own], tbuf)
        pltpu.sync_copy(valid_hbm.at[wid_own], vbuf)

        def w_body(k, _):
            for j in range(C // LANES):
                sl = pl.ds(j * LANES, LANES)
                t = tbuf[k, sl]
                ibuf[k, sl] = dbuf[k, sl] * r + t
                gbuf[k, sl] = t * n + sbuf[k, sl]
            pltpu.sync_copy(acc.at[ibuf.at[k]], cbuf)
            for j in range(C // LANES):
                sl = pl.ds(j * LANES, LANES)
                wbuf[pl.ds(k * C + j * LANES, LANES)] = (
                    vbuf[k, sl] / jnp.maximum(cbuf[sl], 1.0))
            return 0
        lax.fori_loop(0, kc, w_body, 0)

        pltpu.sync_copy(wbuf, w_hbm.at[wid_own])
        pltpu.sync_copy(gbuf, g_hbm.at[wid_own])

    return counts_weights


def _make_edge_pass(npad, d, kc):
    rows_per_tile = npad // NS
    piece = 32
    npiece = rows_per_tile // piece
    peel = 2 - (kc % 2)

    @functools.partial(
        pl.kernel,
        out_type=jax.ShapeDtypeStruct((NC, npad, d), jnp.float32),
        mesh=_sc_mesh(),
        scratch_types=[
            pltpu.VMEM((3, C), jnp.int32),
            pltpu.VMEM((3, C), jnp.int32),
            pltpu.VMEM((C, d), jnp.float32),
            pltpu.VMEM((C, d), jnp.float32),
            pltpu.VMEM((C,), jnp.int32),
            pltpu.VMEM((C,), jnp.int32),
            pltpu.VMEM((piece, d), jnp.float32),
            pltpu.VMEM_SHARED((npad, d), jnp.float32),
            pltpu.SemaphoreType.DMA,
            pltpu.SemaphoreType.DMA,
            pltpu.SemaphoreType.DMA,
            pltpu.SemaphoreType.DMA,
            pltpu.SemaphoreType.DMA,
            pltpu.SemaphoreType.DMA,
        ],
    )
    def edge_pass(h_hbm, edata_hbm, part_hbm, ebuf0, ebuf1, rows0, rows1,
                  di0, di1, zrow, acc, semg0, semg1, sems0, sems1, seme0,
                  seme1):
        cid = lax.axis_index("c")
        sid = lax.axis_index("s")
        wid = cid * NS + sid
        ebufs = (ebuf0, ebuf1)
        rows = (rows0, rows1)
        dis = (di0, di1)
        semg = (semg0, semg1)
        sems = (sems0, sems1)
        seme = (seme0, seme1)

        def z_body(i, _):
            for j in range(d // LANES):
                zrow[i, pl.ds(j * LANES, LANES)] = jnp.zeros((LANES,), jnp.float32)
            return 0
        lax.fori_loop(0, piece, z_body, 0)
        for p in range(npiece):
            pltpu.sync_copy(
                zrow, acc.at[pl.ds(sid * rows_per_tile + p * piece, piece)])
        plsc.subcore_barrier()

        def sub_iter(k, p, first):
            q = 1 - p
            eb, rw, di = ebufs[p], rows[p], dis[p]
            ebq, rwq, diq = ebufs[q], rows[q], dis[q]
            pltpu.make_async_copy(h_hbm.at[eb.at[0]], rw, semg[p]).wait()
            if not first:
                pltpu.make_async_copy(rwq, acc.at[diq], sems[q]).wait()
            pltpu.make_async_copy(edata_hbm.at[wid, k + 1], ebq,
                                  seme[q]).wait()
            pltpu.async_copy(h_hbm.at[ebq.at[0]], rwq, semg[q])
            for g in range(C // LANES):
                sl = pl.ds(g * LANES, LANES)
                di[sl] = eb[1, sl]
                wv = lax.bitcast_convert_type(eb[2, sl], jnp.float32)
                for t in range(LANES):
                    wt = lax.gather(
                        wv, jnp.full((LANES, 1), t, jnp.int32),
                        lax.GatherDimensionNumbers(
                            offset_dims=(), collapsed_slice_dims=(0,),
                            start_index_map=(0,)),
                        slice_sizes=(1,),
                        mode=lax.GatherScatterMode.PROMISE_IN_BOUNDS)
                    e = g * LANES + t
                    for j in range(d // LANES):
                        slj = pl.ds(j * LANES, LANES)
                        rw[e, slj] = rw[e, slj] * wt
            pltpu.async_copy(rw, acc.at[di], sems[p], add=True)
            pltpu.async_copy(edata_hbm.at[wid, k + 2], eb, seme[p])

        pltpu.sync_copy(edata_hbm.at[wid, 0], ebuf0)
        pltpu.async_copy(h_hbm.at[ebuf0.at[0]], rows0, semg0)
        pltpu.async_copy(edata_hbm.at[wid, 1], ebuf1, seme1)
        for k in range(peel):
            sub_iter(k, k % 2, k == 0)

        def body(i, _):
            k = peel + 2 * i
            sub_iter(k, peel % 2, False)
            sub_iter(k + 1, (peel + 1) % 2, False)
            return 0
        lax.fori_loop(0, (kc - peel) // 2, body, 0)

        pg = kc % 2
        pl_ = (kc - 1) % 2
        pltpu.make_async_copy(h_hbm.at[ebufs[pg].at[0]], rows[pg],
                              semg[pg]).wait()
        pltpu.make_async_copy(rows[pl_], acc.at[dis[pl_]], sems[pl_]).wait()
        pltpu.make_async_copy(edata_hbm.at[wid, kc + 1], ebufs[pl_],
                              seme[pl_]).wait()
        plsc.subcore_barrier()

        for p in range(npiece):
            sl = pl.ds(sid * rows_per_tile + p * piece, piece)
            pltpu.sync_copy(acc.at[sl], zrow)
            pltpu.sync_copy(zrow, part_hbm.at[cid, sl])

    return edge_pass


def kernel(x, edge_index, edge_type, W_rel1, W_root1, b1, W_rel2, W_root2,
           b2, W_out, b_out):
    n, d = x.shape
    r = W_rel1.shape[0]
    e = edge_type.shape[0]
    nr = n * r

    step = NW * C
    e_pad = ((e + step - 1) // step) * step
    kc = e_pad // (NW * C)

    src = edge_index[0]
    dst = edge_index[1]
    valid = jnp.ones((e,), jnp.float32)
    if e_pad != e:
        pad = e_pad - e
        src = jnp.concatenate(
            [src, (jnp.arange(pad, dtype=src.dtype) % n)])
        dst = jnp.concatenate([dst, jnp.zeros((pad,), dst.dtype)])
        edge_type = jnp.concatenate([edge_type, jnp.zeros((pad,), edge_type.dtype)])
        valid = jnp.concatenate([valid, jnp.zeros((pad,), jnp.float32)])
    src_r = src.reshape(NW, kc, C)
    dst_r = dst.reshape(NW, kc, C)
    type_r = edge_type.reshape(NW, kc, C)
    valid_r = valid.reshape(NW, kc, C)

    binstep = NS * 1024
    nrp = ((nr + binstep - 1) // binstep) * binstep
    nodestep = NS * 128
    npad = ((n + nodestep - 1) // nodestep) * nodestep

    w_r, g_r = _make_counts_weights(n, nrp, kc, r)(src_r, dst_r, type_r,
                                                   valid_r)

    edata = jnp.stack(
        [g_r, dst_r, lax.bitcast_convert_type(w_r.reshape(NW, kc, C), jnp.int32)],
        axis=2)
    edata = jnp.pad(edata, ((0, 0), (0, 2), (0, 0), (0, 0)))

    edge_pass = _make_edge_pass(npad, d, kc)
    nb = 1000

    w_all1 = jnp.concatenate([W_rel1, W_root1[None]], axis=0)
    h_all1 = _tc_transform(x, w_all1, nb)
    part1 = edge_pass(h_all1[:r].reshape(r * n, d), edata)

    w_all2 = jnp.concatenate([W_rel2, W_root2[None]], axis=0)
    h_all2 = _tc_transform2(h_all1[r], part1[0, :n], part1[1, :n],
                            b1.reshape(1, d), w_all2, nb)
    part2 = edge_pass(h_all2[:r].reshape(r * n, d), edata)
    return _tc_combine2(h_all2[r], part2[0, :n], part2[1, :n],
                        b2.reshape(1, d), W_out, b_out.reshape(1, 1))
